# Initial kernel scaffold; baseline (speedup 1.0000x reference)
import jax
import jax.numpy as jnp
from jax.experimental import pallas as pl


def kernel(tokens, table):
    raise NotImplementedError("write your pallas kernel here")


# speedup vs baseline: 2.9706x; 2.9706x over previous
import jax
import jax.numpy as jnp
from jax import lax
from jax.experimental import pallas as pl
from jax.experimental.pallas import tpu as pltpu
from jax.experimental.pallas import tpu_sc as plsc

NC = 2
NS = 16
NW = NC * NS

EMBED = 128
CHUNK = 128


def _embed_body(tok_hbm, table_hbm, out_hbm, idx_v, buf, sem, n_chunks):
    wid = lax.axis_index("s") * NC + lax.axis_index("c")
    pltpu.sync_copy(tok_hbm.at[wid], idx_v)
    base = wid * (n_chunks * CHUNK)

    def step(j, carry):
        pltpu.async_copy(table_hbm.at[idx_v.at[j]], buf, sem).wait()
        pltpu.sync_copy(buf, out_hbm.at[pl.ds(base + j * CHUNK, CHUNK)])
        return carry

    lax.fori_loop(0, n_chunks, step, 0)


def kernel(tokens, table):
    n_tok = tokens.shape[0] * tokens.shape[1]
    assert n_tok % (NW * CHUNK) == 0
    n_chunks = n_tok // (NW * CHUNK)
    tok_flat = tokens.reshape(NW, n_chunks, CHUNK).astype(jnp.int32)

    mesh = plsc.VectorSubcoreMesh(
        core_axis_name="c", subcore_axis_name="s",
        num_cores=NC, num_subcores=NS)

    def body(tok_hbm, table_hbm, out_hbm, idx_v, buf, sem):
        _embed_body(tok_hbm, table_hbm, out_hbm, idx_v, buf, sem, n_chunks)

    out = pl.kernel(
        body,
        out_type=jax.ShapeDtypeStruct((n_tok, EMBED), jnp.float32),
        mesh=mesh,
        scratch_types=[
            pltpu.VMEM((n_chunks, CHUNK), jnp.int32),
            pltpu.VMEM((CHUNK, EMBED), jnp.float32),
            pltpu.SemaphoreType.DMA,
        ],
    )(tok_flat, table)
    return out.reshape(tokens.shape[0], tokens.shape[1], EMBED)


# speedup vs baseline: 3.3476x; 1.1269x over previous
import jax
import jax.numpy as jnp
from jax import lax
from jax.experimental import pallas as pl
from jax.experimental.pallas import tpu as pltpu
from jax.experimental.pallas import tpu_sc as plsc

NC = 2
NS = 16
NW = NC * NS

EMBED = 128
CHUNK = 128
NBUF = 5


def _embed_body(tok_hbm, table_hbm, out_hbm, idx_v, bufs, gsem, wsem,
                n_chunks):
    wid = lax.axis_index("s") * NC + lax.axis_index("c")
    pltpu.sync_copy(tok_hbm.at[wid], idx_v)
    base = wid * (n_chunks * CHUNK)

    def g_copy(j, b):
        return pltpu.make_async_copy(
            table_hbm.at[idx_v.at[j]], bufs.at[b], gsem.at[b])

    def w_copy(j, b):
        return pltpu.make_async_copy(
            bufs.at[b], out_hbm.at[pl.ds(base + j * CHUNK, CHUNK)],
            wsem.at[b])

    for b in range(NBUF):
        g_copy(b, b).start()

    @pl.loop(0, n_chunks, step=NBUF)
    def _(j0):
        for b in range(NBUF):
            j = j0 + b
            g_copy(j, b).wait()
            w_copy(j, b).start()

            @pl.when(j + NBUF < n_chunks)
            def _():
                w_copy(j, b).wait()
                g_copy(j + NBUF, b).start()

    for b in range(NBUF):
        w_copy(n_chunks - NBUF + b, b).wait()


def kernel(tokens, table):
    n_tok = tokens.shape[0] * tokens.shape[1]
    assert n_tok % (NW * CHUNK) == 0
    n_chunks = n_tok // (NW * CHUNK)
    assert n_chunks % NBUF == 0
    tok_flat = tokens.reshape(NW, n_chunks, CHUNK).astype(jnp.int32)

    mesh = plsc.VectorSubcoreMesh(
        core_axis_name="c", subcore_axis_name="s",
        num_cores=NC, num_subcores=NS)

    def body(tok_hbm, table_hbm, out_hbm, idx_v, bufs, gsem, wsem):
        _embed_body(tok_hbm, table_hbm, out_hbm, idx_v, bufs, gsem, wsem,
                    n_chunks)

    out = pl.kernel(
        body,
        out_type=jax.ShapeDtypeStruct((n_tok, EMBED), jnp.float32),
        mesh=mesh,
        scratch_types=[
            pltpu.VMEM((n_chunks, CHUNK), jnp.int32),
            pltpu.VMEM((NBUF, CHUNK, EMBED), jnp.float32),
            pltpu.SemaphoreType.DMA((NBUF,)),
            pltpu.SemaphoreType.DMA((NBUF,)),
        ],
    )(tok_flat, table)
    return out.reshape(tokens.shape[0], tokens.shape[1], EMBED)


# speedup vs baseline: 5.9690x; 1.7831x over previous
import jax
import jax.numpy as jnp
from jax import lax
from jax.experimental import pallas as pl
from jax.experimental.pallas import tpu as pltpu
from jax.experimental.pallas import tpu_sc as plsc

NC = 2
NS = 16
NW = NC * NS

EMBED = 128
NBUF = 8


def _embed_body(tok_hbm, table_hbm, out_hbm, idx_v, bufs, gsem, wsem,
                seq_per_w, seq_len):
    wid = lax.axis_index("s") * NC + lax.axis_index("c")
    pltpu.sync_copy(tok_hbm.at[wid], idx_v)
    seq0 = wid * seq_per_w

    def g_copy(j, b):
        return pltpu.make_async_copy(
            table_hbm.at[idx_v.at[j, pl.ds(0, seq_len)]], bufs.at[b],
            gsem.at[b])

    def w_copy(j, b):
        return pltpu.make_async_copy(
            bufs.at[b], out_hbm.at[seq0 + j], wsem.at[b])

    for b in range(NBUF):
        g_copy(b, b).start()

    @pl.loop(0, seq_per_w, step=NBUF)
    def _(j0):
        for b in range(NBUF):
            j = j0 + b
            g_copy(j, b).wait()
            w_copy(j, b).start()

            @pl.when(j + NBUF < seq_per_w)
            def _():
                w_copy(j, b).wait()
                g_copy(j + NBUF, b).start()

    for b in range(NBUF):
        w_copy(seq_per_w - NBUF + b, b).wait()


def kernel(tokens, table):
    n_seq, seq_len = tokens.shape
    assert n_seq % NW == 0
    seq_per_w = n_seq // NW
    assert seq_per_w % NBUF == 0
    tok_pad = jnp.zeros((n_seq, 128), jnp.int32)
    tok_pad = lax.dynamic_update_slice(
        tok_pad, tokens.astype(jnp.int32), (0, 0))
    tok_cube = tok_pad.reshape(NW, seq_per_w, 128)

    mesh = plsc.VectorSubcoreMesh(
        core_axis_name="c", subcore_axis_name="s",
        num_cores=NC, num_subcores=NS)

    def body(tok_hbm, table_hbm, out_hbm, idx_v, bufs, gsem, wsem):
        _embed_body(tok_hbm, table_hbm, out_hbm, idx_v, bufs, gsem, wsem,
                    seq_per_w, seq_len)

    out = pl.kernel(
        body,
        out_type=jax.ShapeDtypeStruct((n_seq, seq_len, EMBED), jnp.float32),
        mesh=mesh,
        compiler_params=pltpu.CompilerParams(use_tc_tiling_on_sc=True),
        scratch_types=[
            pltpu.VMEM((seq_per_w, 128), jnp.int32),
            pltpu.VMEM((NBUF, seq_len, EMBED), jnp.float32),
            pltpu.SemaphoreType.DMA((NBUF,)),
            pltpu.SemaphoreType.DMA((NBUF,)),
        ],
    )(tok_cube, table)
    return out
